# baseline (speedup 1.0000x reference)
import functools

import jax
import jax.numpy as jnp
from jax import lax
from jax.experimental import pallas as pl
from jax.experimental.pallas import tpu as pltpu

S, D, H, Hd = 2048, 2048, 16, 128
E, TOPK, M = 64, 8, 1024
EPS = 1e-05
SCALE = 0.08838834764831845

ST = 256
NST = S // ST
QT = 256
NQT = S // QT
BT = 128
NTOT = S * TOPK
NTILES = NTOT // BT
VISITS = NTILES + E
DJ = 512
NJ = D // DJ

_F32 = jnp.float32


def _rms(t, w):
    return t * lax.rsqrt(jnp.mean(t * t, axis=-1, keepdims=True) + EPS) * w


def _fiota(shape, dim):
    return lax.broadcasted_iota(jnp.int32, shape, dim).astype(_F32)


def _dot(a, b, dims):
    return lax.dot_general(a, b, (dims, ((), ())),
                           preferred_element_type=_F32)


def _proj_kernel(x_ref, w_ref, ln_ref, nw_ref, o_ref, *, rope):
    i = pl.program_id(0)
    x = x_ref[...]
    xn = _rms(x, ln_ref[...])
    t = _dot(xn, w_ref[...], ((1,), (1,)))
    if nw_ref is not None:
        t = _rms(t, nw_ref[...])
    if rope:
        half = Hd // 2
        pos = (jnp.float32(i * ST)
               + _fiota((ST, 1), 0))
        j = _fiota((1, Hd), 1)
        jmod = j - jnp.floor(j / half) * half
        inv = jnp.exp(-jnp.log(10000.0) * jmod / half)
        ang = pos * inv
        cosf = jnp.cos(ang)
        sinf = jnp.sin(ang) * jnp.where(j < half, -1.0, 1.0)
        t3 = t.reshape(ST, H, Hd)
        rot = jnp.concatenate([t3[..., half:], t3[..., :half]], axis=-1)
        t3 = t3 * cosf[:, None, :] + rot * sinf[:, None, :]
        t = t3.reshape(ST, D)
    o_ref[...] = t


def _proj(xf, w, ln, nw, rope):
    in_specs = [
        pl.BlockSpec((ST, D), lambda i: (i, 0)),
        pl.BlockSpec((D, D), lambda i: (0, 0)),
        pl.BlockSpec((1, D), lambda i: (0, 0)),
    ]
    args = [xf, w, ln]
    if nw is not None:
        body = functools.partial(_proj_kernel, rope=rope)
        in_specs.append(pl.BlockSpec((1, D), lambda i: (0, 0)))
        args.append(nw)
    else:
        def body(x_ref, w_ref, ln_ref, o_ref):
            _proj_kernel(x_ref, w_ref, ln_ref, None, o_ref, rope=rope)
    return pl.pallas_call(
        body,
        grid=(NST,),
        in_specs=in_specs,
        out_specs=pl.BlockSpec((ST, D), lambda i: (i, 0)),
        out_shape=jax.ShapeDtypeStruct((S, D), _F32),
        compiler_params=pltpu.CompilerParams(
            dimension_semantics=("arbitrary",)),
    )(*args)


def _attn_kernel(q_ref, k_ref, v_ref, o_ref):
    i = pl.program_id(1)
    q = q_ref[...]
    k = k_ref[...]
    s = _dot(q, k, ((1,), (1,))) * SCALE
    row = (jnp.float32(i * QT)
           + _fiota((QT, 1), 0))
    col = _fiota((QT, S), 1)
    s = jnp.where(col <= row, s, jnp.finfo(_F32).min)
    m = jnp.max(s, axis=1, keepdims=True)
    p = jnp.exp(s - m)
    p = p / jnp.sum(p, axis=1, keepdims=True)
    o_ref[...] = _dot(p, v_ref[...], ((1,), (0,)))


def _attn(q, k, v):
    return pl.pallas_call(
        _attn_kernel,
        grid=(H, NQT),
        in_specs=[
            pl.BlockSpec((QT, Hd), lambda h, i: (i, h)),
            pl.BlockSpec((S, Hd), lambda h, i: (0, h)),
            pl.BlockSpec((S, Hd), lambda h, i: (0, h)),
        ],
        out_specs=pl.BlockSpec((QT, Hd), lambda h, i: (i, h)),
        out_shape=jax.ShapeDtypeStruct((S, D), _F32),
        compiler_params=pltpu.CompilerParams(
            dimension_semantics=("arbitrary", "arbitrary")),
    )(q, k, v)


def _post_kernel(x_ref, ctx_ref, ow_ref, pln_ref, gw_ref,
                 x2_ref, x3_ref, lg_ref):
    x2 = x_ref[...] + _dot(ctx_ref[...], ow_ref[...], ((1,), (1,)))
    x3 = _rms(x2, pln_ref[...])
    x2_ref[...] = x2
    x3_ref[...] = x3
    lg_ref[...] = _dot(x3, gw_ref[...], ((1,), (1,)))


def _post(xf, ctx, ow, pln, gw):
    return pl.pallas_call(
        _post_kernel,
        grid=(NST,),
        in_specs=[
            pl.BlockSpec((ST, D), lambda i: (i, 0)),
            pl.BlockSpec((ST, D), lambda i: (i, 0)),
            pl.BlockSpec((D, D), lambda i: (0, 0)),
            pl.BlockSpec((1, D), lambda i: (0, 0)),
            pl.BlockSpec((E, D), lambda i: (0, 0)),
        ],
        out_specs=[
            pl.BlockSpec((ST, D), lambda i: (i, 0)),
            pl.BlockSpec((ST, D), lambda i: (i, 0)),
            pl.BlockSpec((ST, E), lambda i: (i, 0)),
        ],
        out_shape=[
            jax.ShapeDtypeStruct((S, D), _F32),
            jax.ShapeDtypeStruct((S, D), _F32),
            jax.ShapeDtypeStruct((S, E), _F32),
        ],
        compiler_params=pltpu.CompilerParams(
            dimension_semantics=("arbitrary",)),
    )(xf, ctx, ow, pln, gw)


def _route_kernel(lg_ref, pos_ref, wt_ref, meta_ref):
    lg = lg_ref[...]
    mx = jnp.max(lg, axis=1, keepdims=True)
    p = jnp.exp(lg - mx)
    p = p / jnp.sum(p, axis=1, keepdims=True)

    iota_e = _fiota((S, E), 1)
    pw = p
    idx_list, val_list = [], []
    for _ in range(TOPK):
        m = jnp.max(pw, axis=1, keepdims=True)
        cand = jnp.where(pw == m, iota_e, jnp.float32(E))
        sel = jnp.min(cand, axis=1, keepdims=True)
        idx_list.append(sel)
        val_list.append(m)
        pw = jnp.where(iota_e == sel, -1.0, pw)

    r128 = _fiota((BT, BT), 0)
    c128 = _fiota((BT, BT), 1)
    tri = jnp.where(r128 > c128, 1.0, 0.0).astype(jnp.bfloat16)

    csum_list, hist_list = [], []
    for k in range(TOPK):
        ok = jnp.where(iota_e == idx_list[k], 1.0, 0.0)
        carry = jnp.zeros((1, E), _F32)
        blocks = []
        for b in range(S // BT):
            ob = ok[b * BT:(b + 1) * BT, :]
            cs = _dot(tri, ob.astype(jnp.bfloat16), ((1,), (0,))) + carry
            carry = carry + jnp.sum(ob, axis=0, keepdims=True)
            blocks.append(cs)
        csum_list.append(jnp.concatenate(blocks, axis=0))
        hist_list.append(carry)

    hist = functools.reduce(jnp.add, hist_list)
    r64 = _fiota((E, E), 0)
    c64 = _fiota((E, E), 1)
    tri64 = jnp.where(r64 > c64, 1.0, 0.0)
    offsets = _dot(hist, tri64, ((1,), (1,)))

    pos_cols, wt_cols = [], []
    histpre = jnp.zeros((1, E), _F32)
    for k in range(TOPK):
        ok = jnp.where(iota_e == idx_list[k], 1.0, 0.0)
        posk = jnp.sum((csum_list[k] + histpre + offsets) * ok,
                       axis=1, keepdims=True)
        pos_cols.append(posk)
        wt_cols.append(val_list[k])
        histpre = histpre + hist_list[k]

    pos_ref[...] = jnp.concatenate(pos_cols, axis=1)
    wt_ref[...] = jnp.concatenate(wt_cols, axis=1)

    eye = jnp.where(r64 == c64, 1.0, 0.0)
    hist_col = jnp.sum(eye * hist, axis=1, keepdims=True)
    offs_col = _dot(tri64, hist_col, ((1,), (0,)))
    ends_col = offs_col + hist_col
    tf_col = jnp.floor(offs_col / BT)
    tl_col = jnp.floor((ends_col + (BT - 1)) / BT)
    n_col = jnp.where(hist_col > 0, tl_col - tf_col, 0.0)
    voff_col = _dot(tri64, n_col, ((1,), (0,)))
    total = jnp.sum(n_col, axis=0, keepdims=True)
    e_col = _fiota((E, 1), 0)
    e_last = jnp.max(jnp.where(hist_col > 0, e_col, -1.0),
                     axis=0, keepdims=True)

    viota = _fiota((1, VISITS), 1)
    ble = jnp.where(voff_col <= viota, 1.0, 0.0)
    e_row = jnp.sum(ble, axis=0, keepdims=True) - 1.0
    ohve = jnp.where(e_col == e_row, 1.0, 0.0)
    offs_v = jnp.sum(ohve * offs_col, axis=0, keepdims=True)
    ends_v = jnp.sum(ohve * ends_col, axis=0, keepdims=True)
    tf_v = jnp.sum(ohve * tf_col, axis=0, keepdims=True)
    voff_v = jnp.sum(ohve * voff_col, axis=0, keepdims=True)

    valid = viota < total
    tile_v = jnp.where(valid, tf_v + (viota - voff_v),
                       jnp.float32(NTILES - 1))
    g_v = jnp.where(valid, e_row, e_last)
    base_v = tile_v * BT
    rs_v = jnp.where(valid, jnp.maximum(offs_v, base_v), 0.0)
    re_v = jnp.where(valid, jnp.minimum(ends_v, base_v + BT), 0.0)

    meta_ref[0:1, :] = g_v.astype(jnp.int32)
    meta_ref[1:2, :] = tile_v.astype(jnp.int32)
    meta_ref[2:3, :] = rs_v.astype(jnp.int32)
    meta_ref[3:4, :] = re_v.astype(jnp.int32)
    meta_ref[4:8, :] = jnp.zeros((4, VISITS), jnp.int32)


def _route(logits):
    return pl.pallas_call(
        _route_kernel,
        out_shape=[
            jax.ShapeDtypeStruct((S, TOPK), _F32),
            jax.ShapeDtypeStruct((S, TOPK), _F32),
            jax.ShapeDtypeStruct((8, VISITS), jnp.int32),
        ],
    )(logits)


def _gather_kernel(pos_ref, wt_ref, hf_ref, xs_ref, tcol_ref, wcol_ref):
    t = pl.program_id(0)
    rowpos = (jnp.float32(t * BT)
              + _fiota((BT, 1), 0))
    oh = jnp.zeros((BT, S), _F32)
    whw = jnp.zeros((BT, S), _F32)
    for k in range(TOPK):
        cmp = pos_ref[k:k + 1, :] == rowpos
        oh = oh + cmp.astype(_F32)
        whw = whw + jnp.where(cmp, wt_ref[k:k + 1, :], 0.0)
    xs_ref[...] = _dot(oh, hf_ref[...], ((1,), (0,)))
    tio = _fiota((1, S), 1)
    tcol_ref[...] = jnp.sum(oh * tio, axis=1, keepdims=True)
    wcol_ref[...] = jnp.sum(whw, axis=1, keepdims=True)


def _gather(x3, pos_t, wt_t):
    return pl.pallas_call(
        _gather_kernel,
        grid=(NTILES,),
        in_specs=[
            pl.BlockSpec((TOPK, S), lambda t: (0, 0)),
            pl.BlockSpec((TOPK, S), lambda t: (0, 0)),
            pl.BlockSpec((S, D), lambda t: (0, 0)),
        ],
        out_specs=[
            pl.BlockSpec((BT, D), lambda t: (t, 0)),
            pl.BlockSpec((BT, 1), lambda t: (t, 0)),
            pl.BlockSpec((BT, 1), lambda t: (t, 0)),
        ],
        out_shape=[
            jax.ShapeDtypeStruct((NTOT, D), _F32),
            jax.ShapeDtypeStruct((NTOT, 1), _F32),
            jax.ShapeDtypeStruct((NTOT, 1), _F32),
        ],
        compiler_params=pltpu.CompilerParams(
            dimension_semantics=("arbitrary",)),
    )(pos_t, wt_t, x3)


def _gmma_kernel(meta_ref, xs_ref, wg_ref, wu_ref, h_ref):
    v = pl.program_id(0)
    tile = meta_ref[1, v]
    rs = meta_ref[2, v]
    re = meta_ref[3, v]
    x = xs_ref[...]
    a = _dot(x, wg_ref[0], ((1,), (1,)))
    b = _dot(x, wu_ref[0], ((1,), (1,)))
    h = (a * jax.nn.sigmoid(a)) * b
    rowpos = (tile * BT
              + lax.broadcasted_iota(jnp.int32, (BT, 1), 0))
    validf = jnp.where((rowpos >= rs) & (rowpos < re), 1.0, 0.0)
    h = h * validf
    prev = meta_ref[1, jnp.maximum(v - 1, 0)]
    first = (v == 0) | (tile != prev)

    @pl.when(first)
    def _():
        h_ref[...] = h

    @pl.when(jnp.logical_not(first))
    def _():
        h_ref[...] = h_ref[...] + h


def _gmm_a(meta, xs, w_gate, w_up):
    grid_spec = pltpu.PrefetchScalarGridSpec(
        num_scalar_prefetch=1,
        grid=(VISITS,),
        in_specs=[
            pl.BlockSpec((BT, D), lambda v, m: (m[1, v], 0)),
            pl.BlockSpec((1, M, D), lambda v, m: (m[0, v], 0, 0)),
            pl.BlockSpec((1, M, D), lambda v, m: (m[0, v], 0, 0)),
        ],
        out_specs=pl.BlockSpec((BT, M), lambda v, m: (m[1, v], 0)),
    )
    return pl.pallas_call(
        _gmma_kernel,
        grid_spec=grid_spec,
        out_shape=jax.ShapeDtypeStruct((NTOT, M), _F32),
        compiler_params=pltpu.CompilerParams(
            dimension_semantics=("arbitrary",)),
    )(meta, xs, w_gate, w_up)


def _gmmb_kernel(meta_ref, h_ref, wd_ref, y_ref):
    v = pl.program_id(0)
    tile = meta_ref[1, v]
    rs = meta_ref[2, v]
    re = meta_ref[3, v]
    y = _dot(h_ref[...], wd_ref[0], ((1,), (1,)))
    rowpos = (tile * BT
              + lax.broadcasted_iota(jnp.int32, (BT, 1), 0))
    validf = jnp.where((rowpos >= rs) & (rowpos < re), 1.0, 0.0)
    y = y * validf
    prev = meta_ref[1, jnp.maximum(v - 1, 0)]
    first = (v == 0) | (tile != prev)

    @pl.when(first)
    def _():
        y_ref[...] = y

    @pl.when(jnp.logical_not(first))
    def _():
        y_ref[...] = y_ref[...] + y


def _gmm_b(meta, hbuf, w_down):
    grid_spec = pltpu.PrefetchScalarGridSpec(
        num_scalar_prefetch=1,
        grid=(VISITS,),
        in_specs=[
            pl.BlockSpec((BT, M), lambda v, m: (m[1, v], 0)),
            pl.BlockSpec((1, D, M), lambda v, m: (m[0, v], 0, 0)),
        ],
        out_specs=pl.BlockSpec((BT, D), lambda v, m: (m[1, v], 0)),
    )
    return pl.pallas_call(
        _gmmb_kernel,
        grid_spec=grid_spec,
        out_shape=jax.ShapeDtypeStruct((NTOT, D), _F32),
        compiler_params=pltpu.CompilerParams(
            dimension_semantics=("arbitrary",)),
    )(meta, hbuf, w_down)


DJ2 = 1024
NJ2 = D // DJ2


def _combine_kernel(tcol_ref, wcol_ref, y_ref, o_ref, acc_ref):
    t = pl.program_id(1)
    tc = tcol_ref[0]
    wc = wcol_ref[0]
    tio_c = _fiota((S, 1), 0)
    ohwt = jnp.where(tc == tio_c, wc, 0.0)
    contrib = _dot(ohwt, y_ref[...], ((1,), (0,)))

    @pl.when(t == 0)
    def _():
        acc_ref[...] = contrib

    @pl.when(t != 0)
    def _():
        acc_ref[...] = acc_ref[...] + contrib

    @pl.when(t == NTILES - 1)
    def _():
        o_ref[...] = acc_ref[...]


def _combine(tcol, wcol, ybuf):
    return pl.pallas_call(
        _combine_kernel,
        grid=(NJ2, NTILES),
        in_specs=[
            pl.BlockSpec((1, 1, BT), lambda j, t: (t, 0, 0)),
            pl.BlockSpec((1, 1, BT), lambda j, t: (t, 0, 0)),
            pl.BlockSpec((BT, DJ2), lambda j, t: (t, j)),
        ],
        out_specs=pl.BlockSpec((S, DJ2), lambda j, t: (0, j)),
        out_shape=jax.ShapeDtypeStruct((S, D), _F32),
        scratch_shapes=[pltpu.VMEM((S, DJ2), _F32)],
        compiler_params=pltpu.CompilerParams(
            dimension_semantics=("arbitrary", "arbitrary")),
    )(tcol, wcol, ybuf)


def _addres_kernel(a_ref, b_ref, o_ref):
    o_ref[...] = a_ref[...] + b_ref[...]


def _addres(x2, moe):
    return pl.pallas_call(
        _addres_kernel,
        grid=(NST,),
        in_specs=[pl.BlockSpec((ST, D), lambda i: (i, 0)),
                  pl.BlockSpec((ST, D), lambda i: (i, 0))],
        out_specs=pl.BlockSpec((ST, D), lambda i: (i, 0)),
        out_shape=jax.ShapeDtypeStruct((S, D), _F32),
        compiler_params=pltpu.CompilerParams(
            dimension_semantics=("arbitrary",)),
    )(x2, moe)


def kernel(x, input_ln_w, q_w, k_w, v_w, q_norm_w, k_norm_w, o_w,
           post_ln_w, gate_w, w_gate, w_up, w_down):
    xf = x.reshape(S, D)
    ln = input_ln_w.reshape(1, D)
    qnw = q_norm_w.reshape(1, D)
    knw = k_norm_w.reshape(1, D)
    pln = post_ln_w.reshape(1, D)

    q = _proj(xf, q_w, ln, qnw, rope=True)
    k = _proj(xf, k_w, ln, knw, rope=True)
    v = _proj(xf, v_w, ln, None, rope=False)
    ctx = _attn(q, k, v)
    x2, x3, logits = _post(xf, ctx, o_w, pln, gw=gate_w)
    pos2d, wt2d, meta = _route(logits)
    pos_t = pos2d.T
    wt_t = wt2d.T
    xs, tcol, wcol = _gather(x3, pos_t, wt_t)
    hbuf = _gmm_a(meta, xs, w_gate, w_up)
    ybuf = _gmm_b(meta, hbuf, w_down)
    moe = _combine(tcol.reshape(NTILES, 1, BT), wcol.reshape(NTILES, 1, BT),
                   ybuf)
    out = _addres(x2, moe)
    return out.reshape(1, S, D)


# speedup vs baseline: 1.3687x; 1.3687x over previous
import functools

import jax
import jax.numpy as jnp
from jax import lax
from jax.experimental import pallas as pl
from jax.experimental.pallas import tpu as pltpu
from jax.experimental.pallas import tpu_sc as plsc

S, D, H, Hd = 2048, 2048, 16, 128
E, TOPK, M = 64, 8, 1024
EPS = 1e-05
SCALE = 0.08838834764831845

ST = 256
NST = S // ST
QT = 256
NQT = S // QT
BT = 128
NTOT = S * TOPK
NTILES = NTOT // BT
VISITS = NTILES + E
DJ = 512
NJ = D // DJ

_F32 = jnp.float32


def _rms(t, w):
    return t * lax.rsqrt(jnp.mean(t * t, axis=-1, keepdims=True) + EPS) * w


def _fiota(shape, dim):
    return lax.broadcasted_iota(jnp.int32, shape, dim).astype(_F32)


def _dot(a, b, dims):
    return lax.dot_general(a, b, (dims, ((), ())),
                           preferred_element_type=_F32)


def _proj_kernel(x_ref, w_ref, ln_ref, nw_ref, o_ref, *, rope):
    i = pl.program_id(0)
    x = x_ref[...]
    xn = _rms(x, ln_ref[...])
    t = _dot(xn, w_ref[...], ((1,), (1,)))
    if nw_ref is not None:
        t = _rms(t, nw_ref[...])
    if rope:
        half = Hd // 2
        pos = (jnp.float32(i * ST)
               + _fiota((ST, 1), 0))
        j = _fiota((1, Hd), 1)
        jmod = j - jnp.floor(j / half) * half
        inv = jnp.exp(-jnp.log(10000.0) * jmod / half)
        ang = pos * inv
        cosf = jnp.cos(ang)
        sinf = jnp.sin(ang) * jnp.where(j < half, -1.0, 1.0)
        t3 = t.reshape(ST, H, Hd)
        rot = jnp.concatenate([t3[..., half:], t3[..., :half]], axis=-1)
        t3 = t3 * cosf[:, None, :] + rot * sinf[:, None, :]
        t = t3.reshape(ST, D)
    o_ref[...] = t


def _proj(xf, w, ln, nw, rope):
    in_specs = [
        pl.BlockSpec((ST, D), lambda i: (i, 0)),
        pl.BlockSpec((D, D), lambda i: (0, 0)),
        pl.BlockSpec((1, D), lambda i: (0, 0)),
    ]
    args = [xf, w, ln]
    if nw is not None:
        body = functools.partial(_proj_kernel, rope=rope)
        in_specs.append(pl.BlockSpec((1, D), lambda i: (0, 0)))
        args.append(nw)
    else:
        def body(x_ref, w_ref, ln_ref, o_ref):
            _proj_kernel(x_ref, w_ref, ln_ref, None, o_ref, rope=rope)
    return pl.pallas_call(
        body,
        grid=(NST,),
        in_specs=in_specs,
        out_specs=pl.BlockSpec((ST, D), lambda i: (i, 0)),
        out_shape=jax.ShapeDtypeStruct((S, D), _F32),
        compiler_params=pltpu.CompilerParams(
            dimension_semantics=("arbitrary",)),
    )(*args)


def _attn_kernel(q_ref, k_ref, v_ref, o_ref):
    i = pl.program_id(1)
    q = q_ref[...]
    k = k_ref[...]
    s = _dot(q, k, ((1,), (1,))) * SCALE
    row = (jnp.float32(i * QT)
           + _fiota((QT, 1), 0))
    col = _fiota((QT, S), 1)
    s = jnp.where(col <= row, s, jnp.finfo(_F32).min)
    m = jnp.max(s, axis=1, keepdims=True)
    p = jnp.exp(s - m)
    p = p / jnp.sum(p, axis=1, keepdims=True)
    o_ref[...] = _dot(p, v_ref[...], ((1,), (0,)))


def _attn(q, k, v):
    return pl.pallas_call(
        _attn_kernel,
        grid=(H, NQT),
        in_specs=[
            pl.BlockSpec((QT, Hd), lambda h, i: (i, h)),
            pl.BlockSpec((S, Hd), lambda h, i: (0, h)),
            pl.BlockSpec((S, Hd), lambda h, i: (0, h)),
        ],
        out_specs=pl.BlockSpec((QT, Hd), lambda h, i: (i, h)),
        out_shape=jax.ShapeDtypeStruct((S, D), _F32),
        compiler_params=pltpu.CompilerParams(
            dimension_semantics=("arbitrary", "arbitrary")),
    )(q, k, v)


def _post_kernel(x_ref, ctx_ref, ow_ref, pln_ref, gw_ref,
                 x2_ref, x3_ref, lg_ref):
    x2 = x_ref[...] + _dot(ctx_ref[...], ow_ref[...], ((1,), (1,)))
    x3 = _rms(x2, pln_ref[...])
    x2_ref[...] = x2
    x3_ref[...] = x3
    lg_ref[...] = _dot(x3, gw_ref[...], ((1,), (1,)))


def _post(xf, ctx, ow, pln, gw):
    return pl.pallas_call(
        _post_kernel,
        grid=(NST,),
        in_specs=[
            pl.BlockSpec((ST, D), lambda i: (i, 0)),
            pl.BlockSpec((ST, D), lambda i: (i, 0)),
            pl.BlockSpec((D, D), lambda i: (0, 0)),
            pl.BlockSpec((1, D), lambda i: (0, 0)),
            pl.BlockSpec((E, D), lambda i: (0, 0)),
        ],
        out_specs=[
            pl.BlockSpec((ST, D), lambda i: (i, 0)),
            pl.BlockSpec((ST, D), lambda i: (i, 0)),
            pl.BlockSpec((ST, E), lambda i: (i, 0)),
        ],
        out_shape=[
            jax.ShapeDtypeStruct((S, D), _F32),
            jax.ShapeDtypeStruct((S, D), _F32),
            jax.ShapeDtypeStruct((S, E), _F32),
        ],
        compiler_params=pltpu.CompilerParams(
            dimension_semantics=("arbitrary",)),
    )(xf, ctx, ow, pln, gw)


def _route_kernel(lg_ref, pos_ref, wt_ref, meta_ref):
    lg = lg_ref[...]
    mx = jnp.max(lg, axis=1, keepdims=True)
    p = jnp.exp(lg - mx)
    p = p / jnp.sum(p, axis=1, keepdims=True)

    iota_e = _fiota((S, E), 1)
    pw = p
    idx_list, val_list = [], []
    for _ in range(TOPK):
        m = jnp.max(pw, axis=1, keepdims=True)
        cand = jnp.where(pw == m, iota_e, jnp.float32(E))
        sel = jnp.min(cand, axis=1, keepdims=True)
        idx_list.append(sel)
        val_list.append(m)
        pw = jnp.where(iota_e == sel, -1.0, pw)

    r128 = _fiota((BT, BT), 0)
    c128 = _fiota((BT, BT), 1)
    tri = jnp.where(r128 > c128, 1.0, 0.0).astype(jnp.bfloat16)

    csum_list, hist_list = [], []
    for k in range(TOPK):
        ok = jnp.where(iota_e == idx_list[k], 1.0, 0.0)
        carry = jnp.zeros((1, E), _F32)
        blocks = []
        for b in range(S // BT):
            ob = ok[b * BT:(b + 1) * BT, :]
            cs = _dot(tri, ob.astype(jnp.bfloat16), ((1,), (0,))) + carry
            carry = carry + jnp.sum(ob, axis=0, keepdims=True)
            blocks.append(cs)
        csum_list.append(jnp.concatenate(blocks, axis=0))
        hist_list.append(carry)

    hist = functools.reduce(jnp.add, hist_list)
    r64 = _fiota((E, E), 0)
    c64 = _fiota((E, E), 1)
    tri64 = jnp.where(r64 > c64, 1.0, 0.0)
    offsets = _dot(hist, tri64, ((1,), (1,)))

    pos_cols, wt_cols = [], []
    histpre = jnp.zeros((1, E), _F32)
    for k in range(TOPK):
        ok = jnp.where(iota_e == idx_list[k], 1.0, 0.0)
        posk = jnp.sum((csum_list[k] + histpre + offsets) * ok,
                       axis=1, keepdims=True)
        pos_cols.append(posk)
        wt_cols.append(val_list[k])
        histpre = histpre + hist_list[k]

    pos_ref[...] = jnp.concatenate(pos_cols, axis=1)
    wt_ref[...] = jnp.concatenate(wt_cols, axis=1)

    eye = jnp.where(r64 == c64, 1.0, 0.0)
    hist_col = jnp.sum(eye * hist, axis=1, keepdims=True)
    offs_col = _dot(tri64, hist_col, ((1,), (0,)))
    ends_col = offs_col + hist_col
    tf_col = jnp.floor(offs_col / BT)
    tl_col = jnp.floor((ends_col + (BT - 1)) / BT)
    n_col = jnp.where(hist_col > 0, tl_col - tf_col, 0.0)
    voff_col = _dot(tri64, n_col, ((1,), (0,)))
    total = jnp.sum(n_col, axis=0, keepdims=True)
    e_col = _fiota((E, 1), 0)
    e_last = jnp.max(jnp.where(hist_col > 0, e_col, -1.0),
                     axis=0, keepdims=True)

    viota = _fiota((1, VISITS), 1)
    ble = jnp.where(voff_col <= viota, 1.0, 0.0)
    e_row = jnp.sum(ble, axis=0, keepdims=True) - 1.0
    ohve = jnp.where(e_col == e_row, 1.0, 0.0)
    offs_v = jnp.sum(ohve * offs_col, axis=0, keepdims=True)
    ends_v = jnp.sum(ohve * ends_col, axis=0, keepdims=True)
    tf_v = jnp.sum(ohve * tf_col, axis=0, keepdims=True)
    voff_v = jnp.sum(ohve * voff_col, axis=0, keepdims=True)

    valid = viota < total
    tile_v = jnp.where(valid, tf_v + (viota - voff_v),
                       jnp.float32(NTILES - 1))
    g_v = jnp.where(valid, e_row, e_last)
    base_v = tile_v * BT
    rs_v = jnp.where(valid, jnp.maximum(offs_v, base_v), 0.0)
    re_v = jnp.where(valid, jnp.minimum(ends_v, base_v + BT), 0.0)

    meta_ref[0:1, :] = g_v.astype(jnp.int32)
    meta_ref[1:2, :] = tile_v.astype(jnp.int32)
    meta_ref[2:3, :] = rs_v.astype(jnp.int32)
    meta_ref[3:4, :] = re_v.astype(jnp.int32)
    meta_ref[4:8, :] = jnp.zeros((4, VISITS), jnp.int32)


def _route(logits):
    return pl.pallas_call(
        _route_kernel,
        out_shape=[
            jax.ShapeDtypeStruct((S, TOPK), _F32),
            jax.ShapeDtypeStruct((S, TOPK), _F32),
            jax.ShapeDtypeStruct((8, VISITS), jnp.int32),
        ],
    )(logits)


def _gather_kernel(pos_ref, wt_ref, hf_ref, xs_ref, tcol_ref, wcol_ref):
    t = pl.program_id(0)
    rowpos = (jnp.float32(t * BT)
              + _fiota((BT, 1), 0))
    oh = jnp.zeros((BT, S), _F32)
    whw = jnp.zeros((BT, S), _F32)
    for k in range(TOPK):
        cmp = pos_ref[k:k + 1, :] == rowpos
        oh = oh + cmp.astype(_F32)
        whw = whw + jnp.where(cmp, wt_ref[k:k + 1, :], 0.0)
    xs_ref[...] = _dot(oh, hf_ref[...], ((1,), (0,)))
    tio = _fiota((1, S), 1)
    tcol_ref[...] = jnp.sum(oh * tio, axis=1, keepdims=True)
    wcol_ref[...] = jnp.sum(whw, axis=1, keepdims=True)


def _gather(x3, pos_t, wt_t):
    return pl.pallas_call(
        _gather_kernel,
        grid=(NTILES,),
        in_specs=[
            pl.BlockSpec((TOPK, S), lambda t: (0, 0)),
            pl.BlockSpec((TOPK, S), lambda t: (0, 0)),
            pl.BlockSpec((S, D), lambda t: (0, 0)),
        ],
        out_specs=[
            pl.BlockSpec((BT, D), lambda t: (t, 0)),
            pl.BlockSpec((BT, 1), lambda t: (t, 0)),
            pl.BlockSpec((BT, 1), lambda t: (t, 0)),
        ],
        out_shape=[
            jax.ShapeDtypeStruct((NTOT, D), _F32),
            jax.ShapeDtypeStruct((NTOT, 1), _F32),
            jax.ShapeDtypeStruct((NTOT, 1), _F32),
        ],
        compiler_params=pltpu.CompilerParams(
            dimension_semantics=("arbitrary",)),
    )(pos_t, wt_t, x3)


def _sc_dispatch(x3, posm):
    info = plsc.get_sparse_core_info()
    ncores = info.num_cores
    nw = ncores * info.num_subcores
    tpw = S // nw
    ch = 32
    nch = tpw // ch
    mesh = plsc.VectorSubcoreMesh(core_axis_name="c", subcore_axis_name="s")

    @functools.partial(
        pl.kernel, mesh=mesh,
        out_type=jax.ShapeDtypeStruct((NTOT, D), _F32),
        scratch_types=[pltpu.VMEM((ch, D), _F32),
                       pltpu.VMEM((TOPK, ch), jnp.int32),
                       pltpu.SemaphoreType.DMA],
    )
    def disp(x3_hbm, posm_hbm, xs_hbm, rows_v, idx_v, sem):
        wid = lax.axis_index("s") * ncores + lax.axis_index("c")
        for c in range(nch):
            tb = wid * tpw + c * ch
            pltpu.sync_copy(x3_hbm.at[pl.ds(tb, ch)], rows_v)
            for k in range(TOPK):
                pltpu.sync_copy(posm_hbm.at[k, pl.ds(tb, ch)], idx_v.at[k])
            cps = [pltpu.async_copy(rows_v, xs_hbm.at[idx_v.at[k]], sem)
                   for k in range(TOPK)]
            for cp in cps:
                cp.wait()

    return disp(x3, posm)


def _sc_regather(ybuf, posm):
    info = plsc.get_sparse_core_info()
    ncores = info.num_cores
    nw = ncores * info.num_subcores
    tpw = S // nw
    ch = 32
    nch = tpw // ch
    mesh = plsc.VectorSubcoreMesh(core_axis_name="c", subcore_axis_name="s")

    @functools.partial(
        pl.kernel, mesh=mesh,
        out_type=jax.ShapeDtypeStruct((TOPK, S, D), _F32),
        scratch_types=[pltpu.VMEM((ch, D), _F32),
                       pltpu.VMEM((ch,), jnp.int32),
                       pltpu.SemaphoreType.DMA],
    )
    def regather(y_hbm, posm_hbm, yt_hbm, rows_v, idx_v, sem):
        wid = lax.axis_index("s") * ncores + lax.axis_index("c")
        for c in range(nch):
            tb = wid * tpw + c * ch
            for k in range(TOPK):
                pltpu.sync_copy(posm_hbm.at[k, pl.ds(tb, ch)], idx_v)
                pltpu.async_copy(y_hbm.at[idx_v], rows_v, sem).wait()
                pltpu.sync_copy(rows_v, yt_hbm.at[k, pl.ds(tb, ch)])

    return regather(ybuf, posm)


SGT = 128
NSGT = S // SGT


def _segsum_kernel(yt_ref, wt_ref, x2_ref, o_ref):
    acc = x2_ref[...]
    for k in range(TOPK):
        acc = acc + yt_ref[k] * wt_ref[:, k:k + 1]
    o_ref[...] = acc


def _segsum(ytok, wt2d, x2):
    return pl.pallas_call(
        _segsum_kernel,
        grid=(NSGT,),
        in_specs=[
            pl.BlockSpec((TOPK, SGT, D), lambda t: (0, t, 0)),
            pl.BlockSpec((SGT, TOPK), lambda t: (t, 0)),
            pl.BlockSpec((SGT, D), lambda t: (t, 0)),
        ],
        out_specs=pl.BlockSpec((SGT, D), lambda t: (t, 0)),
        out_shape=jax.ShapeDtypeStruct((S, D), _F32),
        compiler_params=pltpu.CompilerParams(
            dimension_semantics=("arbitrary",)),
    )(ytok, wt2d, x2)


def _gmma_kernel(meta_ref, xs_ref, wg_ref, wu_ref, h_ref):
    v = pl.program_id(0)
    tile = meta_ref[1, v]
    rs = meta_ref[2, v]
    re = meta_ref[3, v]
    x = xs_ref[...]
    a = _dot(x, wg_ref[0], ((1,), (1,)))
    b = _dot(x, wu_ref[0], ((1,), (1,)))
    h = (a * jax.nn.sigmoid(a)) * b
    rowpos = (tile * BT
              + lax.broadcasted_iota(jnp.int32, (BT, 1), 0))
    validf = jnp.where((rowpos >= rs) & (rowpos < re), 1.0, 0.0)
    h = h * validf
    prev = meta_ref[1, jnp.maximum(v - 1, 0)]
    first = (v == 0) | (tile != prev)

    @pl.when(first)
    def _():
        h_ref[...] = h

    @pl.when(jnp.logical_not(first))
    def _():
        h_ref[...] = h_ref[...] + h


def _gmm_a(meta, xs, w_gate, w_up):
    grid_spec = pltpu.PrefetchScalarGridSpec(
        num_scalar_prefetch=1,
        grid=(VISITS,),
        in_specs=[
            pl.BlockSpec((BT, D), lambda v, m: (m[1, v], 0)),
            pl.BlockSpec((1, M, D), lambda v, m: (m[0, v], 0, 0)),
            pl.BlockSpec((1, M, D), lambda v, m: (m[0, v], 0, 0)),
        ],
        out_specs=pl.BlockSpec((BT, M), lambda v, m: (m[1, v], 0)),
    )
    return pl.pallas_call(
        _gmma_kernel,
        grid_spec=grid_spec,
        out_shape=jax.ShapeDtypeStruct((NTOT, M), _F32),
        compiler_params=pltpu.CompilerParams(
            dimension_semantics=("arbitrary",)),
    )(meta, xs, w_gate, w_up)


def _gmmb_kernel(meta_ref, h_ref, wd_ref, y_ref):
    v = pl.program_id(0)
    tile = meta_ref[1, v]
    rs = meta_ref[2, v]
    re = meta_ref[3, v]
    y = _dot(h_ref[...], wd_ref[0], ((1,), (1,)))
    rowpos = (tile * BT
              + lax.broadcasted_iota(jnp.int32, (BT, 1), 0))
    validf = jnp.where((rowpos >= rs) & (rowpos < re), 1.0, 0.0)
    y = y * validf
    prev = meta_ref[1, jnp.maximum(v - 1, 0)]
    first = (v == 0) | (tile != prev)

    @pl.when(first)
    def _():
        y_ref[...] = y

    @pl.when(jnp.logical_not(first))
    def _():
        y_ref[...] = y_ref[...] + y


def _gmm_b(meta, hbuf, w_down):
    grid_spec = pltpu.PrefetchScalarGridSpec(
        num_scalar_prefetch=1,
        grid=(VISITS,),
        in_specs=[
            pl.BlockSpec((BT, M), lambda v, m: (m[1, v], 0)),
            pl.BlockSpec((1, D, M), lambda v, m: (m[0, v], 0, 0)),
        ],
        out_specs=pl.BlockSpec((BT, D), lambda v, m: (m[1, v], 0)),
    )
    return pl.pallas_call(
        _gmmb_kernel,
        grid_spec=grid_spec,
        out_shape=jax.ShapeDtypeStruct((NTOT, D), _F32),
        compiler_params=pltpu.CompilerParams(
            dimension_semantics=("arbitrary",)),
    )(meta, hbuf, w_down)


DJ2 = 1024
NJ2 = D // DJ2


def _combine_kernel(tcol_ref, wcol_ref, y_ref, o_ref, acc_ref):
    t = pl.program_id(1)
    tc = tcol_ref[0]
    wc = wcol_ref[0]
    tio_c = _fiota((S, 1), 0)
    ohwt = jnp.where(tc == tio_c, wc, 0.0)
    contrib = _dot(ohwt, y_ref[...], ((1,), (0,)))

    @pl.when(t == 0)
    def _():
        acc_ref[...] = contrib

    @pl.when(t != 0)
    def _():
        acc_ref[...] = acc_ref[...] + contrib

    @pl.when(t == NTILES - 1)
    def _():
        o_ref[...] = acc_ref[...]


def _combine(tcol, wcol, ybuf):
    return pl.pallas_call(
        _combine_kernel,
        grid=(NJ2, NTILES),
        in_specs=[
            pl.BlockSpec((1, 1, BT), lambda j, t: (t, 0, 0)),
            pl.BlockSpec((1, 1, BT), lambda j, t: (t, 0, 0)),
            pl.BlockSpec((BT, DJ2), lambda j, t: (t, j)),
        ],
        out_specs=pl.BlockSpec((S, DJ2), lambda j, t: (0, j)),
        out_shape=jax.ShapeDtypeStruct((S, D), _F32),
        scratch_shapes=[pltpu.VMEM((S, DJ2), _F32)],
        compiler_params=pltpu.CompilerParams(
            dimension_semantics=("arbitrary", "arbitrary")),
    )(tcol, wcol, ybuf)


def _addres_kernel(a_ref, b_ref, o_ref):
    o_ref[...] = a_ref[...] + b_ref[...]


def _addres(x2, moe):
    return pl.pallas_call(
        _addres_kernel,
        grid=(NST,),
        in_specs=[pl.BlockSpec((ST, D), lambda i: (i, 0)),
                  pl.BlockSpec((ST, D), lambda i: (i, 0))],
        out_specs=pl.BlockSpec((ST, D), lambda i: (i, 0)),
        out_shape=jax.ShapeDtypeStruct((S, D), _F32),
        compiler_params=pltpu.CompilerParams(
            dimension_semantics=("arbitrary",)),
    )(x2, moe)


def kernel(x, input_ln_w, q_w, k_w, v_w, q_norm_w, k_norm_w, o_w,
           post_ln_w, gate_w, w_gate, w_up, w_down):
    xf = x.reshape(S, D)
    ln = input_ln_w.reshape(1, D)
    qnw = q_norm_w.reshape(1, D)
    knw = k_norm_w.reshape(1, D)
    pln = post_ln_w.reshape(1, D)

    q = _proj(xf, q_w, ln, qnw, rope=True)
    k = _proj(xf, k_w, ln, knw, rope=True)
    v = _proj(xf, v_w, ln, None, rope=False)
    ctx = _attn(q, k, v)
    x2, x3, logits = _post(xf, ctx, o_w, pln, gw=gate_w)
    pos2d, wt2d, meta = _route(logits)
    posm = pos2d.T.astype(jnp.int32)
    xs = _sc_dispatch(x3, posm)
    hbuf = _gmm_a(meta, xs, w_gate, w_up)
    ybuf = _gmm_b(meta, hbuf, w_down)
    ytok = _sc_regather(ybuf, posm)
    out = _segsum(ytok, wt2d, x2)
    return out.reshape(1, S, D)


# speedup vs baseline: 1.3728x; 1.0030x over previous
import functools

import jax
import jax.numpy as jnp
from jax import lax
from jax.experimental import pallas as pl
from jax.experimental.pallas import tpu as pltpu
from jax.experimental.pallas import tpu_sc as plsc

S, D, H, Hd = 2048, 2048, 16, 128
E, TOPK, M = 64, 8, 1024
EPS = 1e-05
SCALE = 0.08838834764831845

ST = 256
NST = S // ST
QT = 256
NQT = S // QT
BT = 128
NTOT = S * TOPK
NTILES = NTOT // BT
VISITS = NTILES + E

_F32 = jnp.float32


def _rms(t, w):
    return t * lax.rsqrt(jnp.mean(t * t, axis=-1, keepdims=True) + EPS) * w


def _fiota(shape, dim):
    return lax.broadcasted_iota(jnp.int32, shape, dim).astype(_F32)


def _dot(a, b, dims):
    return lax.dot_general(a, b, (dims, ((), ())),
                           preferred_element_type=_F32)


def _proj_kernel(x_ref, w_ref, ln_ref, nw_ref, o_ref, *, rope):
    i = pl.program_id(0)
    x = x_ref[...]
    xn = _rms(x, ln_ref[...])
    t = _dot(xn, w_ref[...], ((1,), (1,)))
    if nw_ref is not None:
        t = _rms(t, nw_ref[...])
    if rope:
        half = Hd // 2
        pos = (jnp.float32(i * ST)
               + _fiota((ST, 1), 0))
        j = _fiota((1, Hd), 1)
        jmod = j - jnp.floor(j / half) * half
        inv = jnp.exp(-jnp.log(10000.0) * jmod / half)
        ang = pos * inv
        cosf = jnp.cos(ang)
        sinf = jnp.sin(ang) * jnp.where(j < half, -1.0, 1.0)
        t3 = t.reshape(ST, H, Hd)
        rot = jnp.concatenate([t3[..., half:], t3[..., :half]], axis=-1)
        t3 = t3 * cosf[:, None, :] + rot * sinf[:, None, :]
        t = t3.reshape(ST, D)
    o_ref[...] = t


def _proj(xf, w, ln, nw, rope):
    in_specs = [
        pl.BlockSpec((ST, D), lambda i: (i, 0)),
        pl.BlockSpec((D, D), lambda i: (0, 0)),
        pl.BlockSpec((1, D), lambda i: (0, 0)),
    ]
    args = [xf, w, ln]
    if nw is not None:
        body = functools.partial(_proj_kernel, rope=rope)
        in_specs.append(pl.BlockSpec((1, D), lambda i: (0, 0)))
        args.append(nw)
    else:
        def body(x_ref, w_ref, ln_ref, o_ref):
            _proj_kernel(x_ref, w_ref, ln_ref, None, o_ref, rope=rope)
    return pl.pallas_call(
        body,
        grid=(NST,),
        in_specs=in_specs,
        out_specs=pl.BlockSpec((ST, D), lambda i: (i, 0)),
        out_shape=jax.ShapeDtypeStruct((S, D), _F32),
        compiler_params=pltpu.CompilerParams(
            dimension_semantics=("arbitrary",)),
    )(*args)


def _attn_kernel(q_ref, k_ref, v_ref, o_ref):
    i = pl.program_id(1)
    q = q_ref[...]
    k = k_ref[...]
    s = _dot(q, k, ((1,), (1,))) * SCALE
    row = (jnp.float32(i * QT)
           + _fiota((QT, 1), 0))
    col = _fiota((QT, S), 1)
    s = jnp.where(col <= row, s, jnp.finfo(_F32).min)
    m = jnp.max(s, axis=1, keepdims=True)
    p = jnp.exp(s - m)
    p = p / jnp.sum(p, axis=1, keepdims=True)
    o_ref[...] = _dot(p, v_ref[...], ((1,), (0,)))


def _attn(q, k, v):
    return pl.pallas_call(
        _attn_kernel,
        grid=(H, NQT),
        in_specs=[
            pl.BlockSpec((QT, Hd), lambda h, i: (i, h)),
            pl.BlockSpec((S, Hd), lambda h, i: (0, h)),
            pl.BlockSpec((S, Hd), lambda h, i: (0, h)),
        ],
        out_specs=pl.BlockSpec((QT, Hd), lambda h, i: (i, h)),
        out_shape=jax.ShapeDtypeStruct((S, D), _F32),
        compiler_params=pltpu.CompilerParams(
            dimension_semantics=("arbitrary", "arbitrary")),
    )(q, k, v)


def _post_kernel(x_ref, ctx_ref, ow_ref, pln_ref, gw_ref,
                 x2_ref, x3_ref, lg_ref):
    x2 = x_ref[...] + _dot(ctx_ref[...], ow_ref[...], ((1,), (1,)))
    x3 = _rms(x2, pln_ref[...])
    x2_ref[...] = x2
    x3_ref[...] = x3
    lg_ref[...] = _dot(x3, gw_ref[...], ((1,), (1,)))


def _post(xf, ctx, ow, pln, gw):
    return pl.pallas_call(
        _post_kernel,
        grid=(NST,),
        in_specs=[
            pl.BlockSpec((ST, D), lambda i: (i, 0)),
            pl.BlockSpec((ST, D), lambda i: (i, 0)),
            pl.BlockSpec((D, D), lambda i: (0, 0)),
            pl.BlockSpec((1, D), lambda i: (0, 0)),
            pl.BlockSpec((E, D), lambda i: (0, 0)),
        ],
        out_specs=[
            pl.BlockSpec((ST, D), lambda i: (i, 0)),
            pl.BlockSpec((ST, D), lambda i: (i, 0)),
            pl.BlockSpec((ST, E), lambda i: (i, 0)),
        ],
        out_shape=[
            jax.ShapeDtypeStruct((S, D), _F32),
            jax.ShapeDtypeStruct((S, D), _F32),
            jax.ShapeDtypeStruct((S, E), _F32),
        ],
        compiler_params=pltpu.CompilerParams(
            dimension_semantics=("arbitrary",)),
    )(xf, ctx, ow, pln, gw)


def _route_kernel(lg_ref, pos_ref, wt_ref, meta_ref):
    lg = lg_ref[...]
    mx = jnp.max(lg, axis=1, keepdims=True)
    p = jnp.exp(lg - mx)
    p = p / jnp.sum(p, axis=1, keepdims=True)

    iota_e = _fiota((S, E), 1)
    pw = p
    idx_list, val_list = [], []
    for _ in range(TOPK):
        m = jnp.max(pw, axis=1, keepdims=True)
        cand = jnp.where(pw == m, iota_e, jnp.float32(E))
        sel = jnp.min(cand, axis=1, keepdims=True)
        idx_list.append(sel)
        val_list.append(m)
        pw = jnp.where(iota_e == sel, -1.0, pw)

    r128 = _fiota((BT, BT), 0)
    c128 = _fiota((BT, BT), 1)
    tri = jnp.where(r128 > c128, 1.0, 0.0).astype(jnp.bfloat16)

    csum_list, hist_list = [], []
    for k in range(TOPK):
        ok = jnp.where(iota_e == idx_list[k], 1.0, 0.0)
        carry = jnp.zeros((1, E), _F32)
        blocks = []
        for b in range(S // BT):
            ob = ok[b * BT:(b + 1) * BT, :]
            cs = _dot(tri, ob.astype(jnp.bfloat16), ((1,), (0,))) + carry
            carry = carry + jnp.sum(ob, axis=0, keepdims=True)
            blocks.append(cs)
        csum_list.append(jnp.concatenate(blocks, axis=0))
        hist_list.append(carry)

    hist = functools.reduce(jnp.add, hist_list)
    r64 = _fiota((E, E), 0)
    c64 = _fiota((E, E), 1)
    tri64 = jnp.where(r64 > c64, 1.0, 0.0)
    offsets = _dot(hist, tri64, ((1,), (1,)))

    pos_cols, wt_cols = [], []
    histpre = jnp.zeros((1, E), _F32)
    for k in range(TOPK):
        ok = jnp.where(iota_e == idx_list[k], 1.0, 0.0)
        posk = jnp.sum((csum_list[k] + histpre + offsets) * ok,
                       axis=1, keepdims=True)
        pos_cols.append(posk)
        wt_cols.append(val_list[k])
        histpre = histpre + hist_list[k]

    pos_ref[...] = jnp.concatenate(pos_cols, axis=1)
    wt_ref[...] = jnp.concatenate(wt_cols, axis=1)

    eye = jnp.where(r64 == c64, 1.0, 0.0)
    hist_col = jnp.sum(eye * hist, axis=1, keepdims=True)
    offs_col = _dot(tri64, hist_col, ((1,), (0,)))
    ends_col = offs_col + hist_col
    tf_col = jnp.floor(offs_col / BT)
    tl_col = jnp.floor((ends_col + (BT - 1)) / BT)
    n_col = jnp.where(hist_col > 0, tl_col - tf_col, 0.0)
    voff_col = _dot(tri64, n_col, ((1,), (0,)))
    total = jnp.sum(n_col, axis=0, keepdims=True)
    e_col = _fiota((E, 1), 0)
    e_last = jnp.max(jnp.where(hist_col > 0, e_col, -1.0),
                     axis=0, keepdims=True)

    viota = _fiota((1, VISITS), 1)
    ble = jnp.where(voff_col <= viota, 1.0, 0.0)
    e_row = jnp.sum(ble, axis=0, keepdims=True) - 1.0
    ohve = jnp.where(e_col == e_row, 1.0, 0.0)
    offs_v = jnp.sum(ohve * offs_col, axis=0, keepdims=True)
    ends_v = jnp.sum(ohve * ends_col, axis=0, keepdims=True)
    tf_v = jnp.sum(ohve * tf_col, axis=0, keepdims=True)
    voff_v = jnp.sum(ohve * voff_col, axis=0, keepdims=True)

    valid = viota < total
    tile_v = jnp.where(valid, tf_v + (viota - voff_v),
                       jnp.float32(NTILES - 1))
    g_v = jnp.where(valid, e_row, e_last)
    base_v = tile_v * BT
    rs_v = jnp.where(valid, jnp.maximum(offs_v, base_v), 0.0)
    re_v = jnp.where(valid, jnp.minimum(ends_v, base_v + BT), 0.0)

    meta_ref[0:1, :] = g_v.astype(jnp.int32)
    meta_ref[1:2, :] = tile_v.astype(jnp.int32)
    meta_ref[2:3, :] = rs_v.astype(jnp.int32)
    meta_ref[3:4, :] = re_v.astype(jnp.int32)
    meta_ref[4:8, :] = jnp.zeros((4, VISITS), jnp.int32)


def _route(logits):
    return pl.pallas_call(
        _route_kernel,
        out_shape=[
            jax.ShapeDtypeStruct((S, TOPK), _F32),
            jax.ShapeDtypeStruct((S, TOPK), _F32),
            jax.ShapeDtypeStruct((8, VISITS), jnp.int32),
        ],
    )(logits)


def _sc_dispatch(x3, posm):
    info = plsc.get_sparse_core_info()
    ncores = info.num_cores
    nw = ncores * info.num_subcores
    tpw = S // nw
    ch = 32
    nch = tpw // ch
    mesh = plsc.VectorSubcoreMesh(core_axis_name="c", subcore_axis_name="s")

    @functools.partial(
        pl.kernel, mesh=mesh,
        out_type=jax.ShapeDtypeStruct((NTOT, D), _F32),
        scratch_types=[pltpu.VMEM((ch, D), _F32),
                       pltpu.VMEM((TOPK, ch), jnp.int32),
                       pltpu.SemaphoreType.DMA],
    )
    def disp(x3_hbm, posm_hbm, xs_hbm, rows_v, idx_v, sem):
        wid = lax.axis_index("s") * ncores + lax.axis_index("c")
        for c in range(nch):
            tb = wid * tpw + c * ch
            pltpu.sync_copy(x3_hbm.at[pl.ds(tb, ch)], rows_v)
            for k in range(TOPK):
                pltpu.sync_copy(posm_hbm.at[k, pl.ds(tb, ch)], idx_v.at[k])
            cps = [pltpu.async_copy(rows_v, xs_hbm.at[idx_v.at[k]], sem)
                   for k in range(TOPK)]
            for cp in cps:
                cp.wait()

    return disp(x3, posm)


def _sc_regather(ybuf, posm):
    info = plsc.get_sparse_core_info()
    ncores = info.num_cores
    nw = ncores * info.num_subcores
    tpw = S // nw
    ch = 32
    nch = tpw // ch
    mesh = plsc.VectorSubcoreMesh(core_axis_name="c", subcore_axis_name="s")

    @functools.partial(
        pl.kernel, mesh=mesh,
        out_type=jax.ShapeDtypeStruct((TOPK, S, D), _F32),
        scratch_types=[pltpu.VMEM((ch, D), _F32),
                       pltpu.VMEM((ch,), jnp.int32),
                       pltpu.SemaphoreType.DMA],
    )
    def regather(y_hbm, posm_hbm, yt_hbm, rows_v, idx_v, sem):
        wid = lax.axis_index("s") * ncores + lax.axis_index("c")
        for c in range(nch):
            tb = wid * tpw + c * ch
            for k in range(TOPK):
                pltpu.sync_copy(posm_hbm.at[k, pl.ds(tb, ch)], idx_v)
                pltpu.async_copy(y_hbm.at[idx_v], rows_v, sem).wait()
                pltpu.sync_copy(rows_v, yt_hbm.at[k, pl.ds(tb, ch)])

    return regather(ybuf, posm)


SGT = 128
NSGT = S // SGT


def _segsum_kernel(yt_ref, wt_ref, x2_ref, o_ref):
    acc = x2_ref[...]
    for k in range(TOPK):
        acc = acc + yt_ref[k] * wt_ref[:, k:k + 1]
    o_ref[...] = acc


def _segsum(ytok, wt2d, x2):
    return pl.pallas_call(
        _segsum_kernel,
        grid=(NSGT,),
        in_specs=[
            pl.BlockSpec((TOPK, SGT, D), lambda t: (0, t, 0)),
            pl.BlockSpec((SGT, TOPK), lambda t: (t, 0)),
            pl.BlockSpec((SGT, D), lambda t: (t, 0)),
        ],
        out_specs=pl.BlockSpec((SGT, D), lambda t: (t, 0)),
        out_shape=jax.ShapeDtypeStruct((S, D), _F32),
        compiler_params=pltpu.CompilerParams(
            dimension_semantics=("arbitrary",)),
    )(ytok, wt2d, x2)


def _gmma_kernel(meta_ref, xs_ref, wg_ref, wu_ref, h_ref):
    v = pl.program_id(0)
    tile = meta_ref[1, v]
    rs = meta_ref[2, v]
    re = meta_ref[3, v]
    x = xs_ref[...]
    a = _dot(x, wg_ref[0], ((1,), (1,)))
    b = _dot(x, wu_ref[0], ((1,), (1,)))
    h = (a * jax.nn.sigmoid(a)) * b
    rowpos = (tile * BT
              + lax.broadcasted_iota(jnp.int32, (BT, 1), 0))
    validf = jnp.where((rowpos >= rs) & (rowpos < re), 1.0, 0.0)
    h = h * validf
    prev = meta_ref[1, jnp.maximum(v - 1, 0)]
    first = (v == 0) | (tile != prev)

    @pl.when(first)
    def _():
        h_ref[...] = h

    @pl.when(jnp.logical_not(first))
    def _():
        h_ref[...] = h_ref[...] + h


def _gmm_a(meta, xs, w_gate, w_up):
    grid_spec = pltpu.PrefetchScalarGridSpec(
        num_scalar_prefetch=1,
        grid=(VISITS,),
        in_specs=[
            pl.BlockSpec((BT, D), lambda v, m: (m[1, v], 0)),
            pl.BlockSpec((1, M, D), lambda v, m: (m[0, v], 0, 0)),
            pl.BlockSpec((1, M, D), lambda v, m: (m[0, v], 0, 0)),
        ],
        out_specs=pl.BlockSpec((BT, M), lambda v, m: (m[1, v], 0)),
    )
    return pl.pallas_call(
        _gmma_kernel,
        grid_spec=grid_spec,
        out_shape=jax.ShapeDtypeStruct((NTOT, M), _F32),
        compiler_params=pltpu.CompilerParams(
            dimension_semantics=("arbitrary",)),
    )(meta, xs, w_gate, w_up)


def _gmmb_kernel(meta_ref, h_ref, wd_ref, y_ref):
    v = pl.program_id(0)
    tile = meta_ref[1, v]
    rs = meta_ref[2, v]
    re = meta_ref[3, v]
    y = _dot(h_ref[...], wd_ref[0], ((1,), (1,)))
    rowpos = (tile * BT
              + lax.broadcasted_iota(jnp.int32, (BT, 1), 0))
    validf = jnp.where((rowpos >= rs) & (rowpos < re), 1.0, 0.0)
    y = y * validf
    prev = meta_ref[1, jnp.maximum(v - 1, 0)]
    first = (v == 0) | (tile != prev)

    @pl.when(first)
    def _():
        y_ref[...] = y

    @pl.when(jnp.logical_not(first))
    def _():
        y_ref[...] = y_ref[...] + y


def _gmm_b(meta, hbuf, w_down):
    grid_spec = pltpu.PrefetchScalarGridSpec(
        num_scalar_prefetch=1,
        grid=(VISITS,),
        in_specs=[
            pl.BlockSpec((BT, M), lambda v, m: (m[1, v], 0)),
            pl.BlockSpec((1, D, M), lambda v, m: (m[0, v], 0, 0)),
        ],
        out_specs=pl.BlockSpec((BT, D), lambda v, m: (m[1, v], 0)),
    )
    return pl.pallas_call(
        _gmmb_kernel,
        grid_spec=grid_spec,
        out_shape=jax.ShapeDtypeStruct((NTOT, D), _F32),
        compiler_params=pltpu.CompilerParams(
            dimension_semantics=("arbitrary",)),
    )(meta, hbuf, w_down)


def kernel(x, input_ln_w, q_w, k_w, v_w, q_norm_w, k_norm_w, o_w,
           post_ln_w, gate_w, w_gate, w_up, w_down):
    xf = x.reshape(S, D)
    ln = input_ln_w.reshape(1, D)
    qnw = q_norm_w.reshape(1, D)
    knw = k_norm_w.reshape(1, D)
    pln = post_ln_w.reshape(1, D)

    q = _proj(xf, q_w, ln, qnw, rope=True)
    k = _proj(xf, k_w, ln, knw, rope=True)
    v = _proj(xf, v_w, ln, None, rope=False)
    ctx = _attn(q, k, v)
    x2, x3, logits = _post(xf, ctx, o_w, pln, gw=gate_w)
    pos2d, wt2d, meta = _route(logits)
    posm = pos2d.T.astype(jnp.int32)
    xs = _sc_dispatch(x3, posm)
    hbuf = _gmm_a(meta, xs, w_gate, w_up)
    ybuf = _gmm_b(meta, hbuf, w_down)
    ytok = _sc_regather(ybuf, posm)
    out = _segsum(ytok, wt2d, x2)
    return out.reshape(1, S, D)
